# baseline (speedup 1.0000x reference)
import functools

import jax
import jax.numpy as jnp
from jax import lax
from jax.experimental import pallas as pl
from jax.experimental.pallas import tpu as pltpu
from jax.experimental.pallas import tpu_sc as plsc

N = 10000
E = 320000
NC = 2
NS = 16
NW = NC * NS
EW = E // NW
C = 128
CHUNKS = 80
EWP = C * CHUNKS
C2 = 80
CHUNKS2 = EWP // C2
NGRP2 = CHUNKS2 // 4
PAD = EWP - EW
NBUF = 2
NGRP = CHUNKS // NBUF
N_ACC = N
RPS = 632
RPS_LAST = N - 15 * RPS

_MESH = plsc.VectorSubcoreMesh(core_axis_name="c", subcore_axis_name="s")


def _init_acc(sid, zeros_hbm, acc_sh):
    @pl.when(sid < NS - 1)
    def _():
        pltpu.sync_copy(zeros_hbm, acc_sh.at[pl.ds(sid * RPS, RPS)])

    @pl.when(sid == NS - 1)
    def _():
        pltpu.sync_copy(zeros_hbm.at[pl.ds(0, RPS_LAST)],
                        acc_sh.at[pl.ds(sid * RPS, RPS_LAST)])


def _copy_out(cid, sid, acc_sh, out_hbm):
    @pl.when(sid < NS - 1)
    def _():
        pltpu.sync_copy(acc_sh.at[pl.ds(sid * RPS, RPS)],
                        out_hbm.at[cid, pl.ds(sid * RPS, RPS)])

    @pl.when(sid == NS - 1)
    def _():
        pltpu.sync_copy(acc_sh.at[pl.ds(sid * RPS, RPS_LAST)],
                        out_hbm.at[cid, pl.ds(sid * RPS, RPS_LAST)])


@functools.partial(
    pl.kernel,
    out_type=jax.ShapeDtypeStruct((NC, N, 128), jnp.float32),
    mesh=_MESH,
    scratch_types=[
        [pltpu.VMEM((C2,), jnp.int32)] * 4,
        [pltpu.VMEM((C2,), jnp.int32)] * 4,
        [pltpu.VMEM((C2, 128), jnp.float32)] * 4,
        pltpu.VMEM_SHARED((N_ACC, 128), jnp.float32),
        [pltpu.SemaphoreType.DMA] * 4,
        [pltpu.SemaphoreType.DMA] * 4,
        [pltpu.SemaphoreType.DMA] * 4,
    ],
)
def _scatter_kernel(y_hbm, src_hbm, dst_hbm, zeros_hbm, out_hbm,
                    srcv, dstv, rows, acc_sh, isem, gsem, ssem):
    cid = lax.axis_index("c")
    sid = lax.axis_index("s")
    wid = sid * NC + cid
    ebase = wid * EWP

    def idx_async(c, b):
        base = ebase + c * C2
        pltpu.async_copy(src_hbm.at[pl.ds(base, C2)], srcv[b], isem[b])
        pltpu.async_copy(dst_hbm.at[pl.ds(base, C2)], dstv[b], isem[b])

    def idx_wait(c, b):
        base = ebase + c * C2
        pltpu.make_async_copy(src_hbm.at[pl.ds(base, C2)], srcv[b], isem[b]).wait()
        pltpu.make_async_copy(dst_hbm.at[pl.ds(base, C2)], dstv[b], isem[b]).wait()

    def gather(b):
        pltpu.async_copy(y_hbm.at[srcv[b]], rows[b], gsem[b])

    def gather_wait(b):
        pltpu.make_async_copy(y_hbm.at[srcv[b]], rows[b], gsem[b]).wait()

    def scatter(b):
        pltpu.async_copy(rows[b], acc_sh.at[dstv[b]], ssem[b], add=True)

    def scatter_wait(b):
        pltpu.make_async_copy(rows[b], acc_sh.at[dstv[b]], ssem[b]).wait()

    _init_acc(sid, zeros_hbm, acc_sh)
    for b in range(4):
        idx_async(b, b)
    for b in range(4):
        idx_wait(b, b)
        gather(b)
    plsc.subcore_barrier()

    def body(g, _):
        cn = 4 * g
        for b in range(4):
            gather_wait(b)
            scatter(b)
        for b in range(4):
            scatter_wait(b)
            idx_async(cn + b, b)
        for b in range(4):
            idx_wait(cn + b, b)
            gather(b)
        return 0

    lax.fori_loop(1, NGRP2, body, 0)
    for b in range(4):
        gather_wait(b)
        scatter(b)
    for b in range(4):
        scatter_wait(b)
    plsc.subcore_barrier()
    _copy_out(cid, sid, acc_sh, out_hbm)


@functools.partial(
    pl.kernel,
    out_type=jax.ShapeDtypeStruct((NC, N, 128), jnp.float32),
    mesh=_MESH,
    scratch_types=[
        pltpu.VMEM((C,), jnp.int32), pltpu.VMEM((C,), jnp.int32),
        pltpu.VMEM((C, 128), jnp.float32),
        pltpu.VMEM_SHARED((N_ACC, 128), jnp.float32),
        pltpu.SemaphoreType.DMA, pltpu.SemaphoreType.DMA,
    ],
)
def _deg_kernel(dst_hbm, ones_hbm, zeros_hbm, out_hbm,
                dv0, dv1, ones_v, acc_sh, s0, s1):
    cid = lax.axis_index("c")
    sid = lax.axis_index("s")
    wid = sid * NC + cid
    ebase = wid * EWP

    dstv = (dv0, dv1)
    ssem = (s0, s1)

    def load(c, b):
        pltpu.sync_copy(dst_hbm.at[pl.ds(ebase + c * C, C)], dstv[b])

    def scatter(b):
        pltpu.async_copy(ones_v, acc_sh.at[dstv[b]], ssem[b], add=True)

    def scatter_wait(b):
        pltpu.make_async_copy(ones_v, acc_sh.at[dstv[b]], ssem[b]).wait()

    _init_acc(sid, zeros_hbm, acc_sh)
    pltpu.sync_copy(ones_hbm, ones_v)
    for b in range(2):
        load(b, b)
    plsc.subcore_barrier()
    for b in range(2):
        scatter(b)

    def body(g, _):
        cn = 2 * g
        for b in range(2):
            scatter_wait(b)
            load(cn + b, b)
            scatter(b)
        return 0

    lax.fori_loop(1, CHUNKS // 2, body, 0)
    for b in range(2):
        scatter_wait(b)
    plsc.subcore_barrier()
    _copy_out(cid, sid, acc_sh, out_hbm)


_ROWS = 1000
_GRID = N // _ROWS


def _elu(v):
    return jnp.where(v > 0, v, jnp.exp(jnp.minimum(v, 0.0)) - 1.0)


def _t1_body(x_ref, w_ref, dinv_ref, y_ref):
    y_ref[...] = jnp.dot(x_ref[...], w_ref[...],
                         preferred_element_type=jnp.float32) * dinv_ref[...]


def _t1(x, W1, dinv):
    return pl.pallas_call(
        _t1_body,
        grid=(_GRID,),
        in_specs=[
            pl.BlockSpec((_ROWS, 128), lambda i: (i, 0)),
            pl.BlockSpec((128, 128), lambda i: (0, 0)),
            pl.BlockSpec((_ROWS, 1), lambda i: (i, 0)),
        ],
        out_specs=pl.BlockSpec((_ROWS, 128), lambda i: (i, 0)),
        out_shape=jax.ShapeDtypeStruct((N, 128), jnp.float32),
    )(x, W1, dinv)


def _t2_body(p0_ref, p1_ref, y_ref, gate_ref, dinv_ref, b_ref, w_ref, o_ref):
    agg = p0_ref[...] + p1_ref[...] + y_ref[...] * gate_ref[...]
    h = _elu(agg * dinv_ref[...] + b_ref[...])
    o_ref[...] = jnp.dot(h, w_ref[...],
                         preferred_element_type=jnp.float32) * dinv_ref[...]


def _t2(p0, p1, y1, gate, dinv, b1, W2):
    return pl.pallas_call(
        _t2_body,
        grid=(_GRID,),
        in_specs=[
            pl.BlockSpec((_ROWS, 128), lambda i: (i, 0)),
            pl.BlockSpec((_ROWS, 128), lambda i: (i, 0)),
            pl.BlockSpec((_ROWS, 128), lambda i: (i, 0)),
            pl.BlockSpec((_ROWS, 1), lambda i: (i, 0)),
            pl.BlockSpec((_ROWS, 1), lambda i: (i, 0)),
            pl.BlockSpec((1, 128), lambda i: (0, 0)),
            pl.BlockSpec((128, 128), lambda i: (0, 0)),
        ],
        out_specs=pl.BlockSpec((_ROWS, 128), lambda i: (i, 0)),
        out_shape=jax.ShapeDtypeStruct((N, 128), jnp.float32),
    )(p0, p1, y1, gate, dinv, b1, W2)


def _t3_body(p0_ref, p1_ref, y_ref, gate_ref, dinv_ref, b_ref, w_ref, bl_ref, o_ref):
    agg = p0_ref[...] + p1_ref[...] + y_ref[...] * gate_ref[...]
    h = _elu(agg * dinv_ref[...] + b_ref[...])
    o_ref[...] = jnp.dot(h, w_ref[...],
                         preferred_element_type=jnp.float32) + bl_ref[...]


def _t3(p0, p1, y2, gate, dinv, b2, Wl, bl):
    return pl.pallas_call(
        _t3_body,
        grid=(_GRID,),
        in_specs=[
            pl.BlockSpec((_ROWS, 128), lambda i: (i, 0)),
            pl.BlockSpec((_ROWS, 128), lambda i: (i, 0)),
            pl.BlockSpec((_ROWS, 128), lambda i: (i, 0)),
            pl.BlockSpec((_ROWS, 1), lambda i: (i, 0)),
            pl.BlockSpec((_ROWS, 1), lambda i: (i, 0)),
            pl.BlockSpec((1, 128), lambda i: (0, 0)),
            pl.BlockSpec((128, 64), lambda i: (0, 0)),
            pl.BlockSpec((1, 64), lambda i: (0, 0)),
        ],
        out_specs=pl.BlockSpec((_ROWS, 64), lambda i: (i, 0)),
        out_shape=jax.ShapeDtypeStruct((N, 64), jnp.float32),
    )(p0, p1, y2, gate, dinv, b2, Wl, bl)


def kernel(x, edge_index, W1, b1, W2, b2, Wl, bl):
    src = edge_index[0].astype(jnp.int32)
    dst = edge_index[1].astype(jnp.int32)
    pad_dst = jnp.arange(NW * PAD, dtype=jnp.int32).reshape(NW, PAD)
    srcp = jnp.concatenate([src.reshape(NW, EW), pad_dst], axis=1).reshape(NW * EWP)
    dstp = jnp.concatenate([dst.reshape(NW, EW), pad_dst], axis=1).reshape(NW * EWP)
    ones = jnp.ones((C, 128), jnp.float32)
    zeros = jnp.zeros((RPS, 128), jnp.float32)

    degp = _deg_kernel(dstp, ones, zeros)
    deg = degp[0, :, 0] + degp[1, :, 0] + 1.0
    deg = deg - (jnp.arange(N, dtype=jnp.float32) < NW * PAD)
    dinv = lax.rsqrt(deg).reshape(N, 1)
    gate = (jnp.arange(N) >= NW * PAD).astype(jnp.float32).reshape(N, 1)

    y1 = _t1(x, W1, dinv)
    p1 = _scatter_kernel(y1, srcp, dstp, zeros)
    y2 = _t2(p1[0], p1[1], y1, gate, dinv, b1.reshape(1, 128), W2)
    p2 = _scatter_kernel(y2, srcp, dstp, zeros)
    out = _t3(p2[0], p2[1], y2, gate, dinv, b2.reshape(1, 128), Wl,
              bl.reshape(1, 64))
    return out


# speedup vs baseline: 1.0517x; 1.0517x over previous
import functools

import jax
import jax.numpy as jnp
from jax import lax
from jax.experimental import pallas as pl
from jax.experimental.pallas import tpu as pltpu
from jax.experimental.pallas import tpu_sc as plsc

N = 10000
E = 320000
NC = 2
NS = 16
NW = NC * NS
EW = E // NW
C = 128
CHUNKS = 80
EWP = C * CHUNKS
C2 = 40
RS = 8
CHUNKS2 = EWP // C2
NGRP2 = CHUNKS2 // RS
PAD = EWP - EW
NBUF = 2
NGRP = CHUNKS // NBUF
N_ACC = N
RPS = 632
RPS_LAST = N - 15 * RPS

_MESH = plsc.VectorSubcoreMesh(core_axis_name="c", subcore_axis_name="s")


def _init_acc(sid, zeros_hbm, acc_sh):
    @pl.when(sid < NS - 1)
    def _():
        pltpu.sync_copy(zeros_hbm, acc_sh.at[pl.ds(sid * RPS, RPS)])

    @pl.when(sid == NS - 1)
    def _():
        pltpu.sync_copy(zeros_hbm.at[pl.ds(0, RPS_LAST)],
                        acc_sh.at[pl.ds(sid * RPS, RPS_LAST)])


def _copy_out(cid, sid, acc_sh, out_hbm):
    @pl.when(sid < NS - 1)
    def _():
        pltpu.sync_copy(acc_sh.at[pl.ds(sid * RPS, RPS)],
                        out_hbm.at[cid, pl.ds(sid * RPS, RPS)])

    @pl.when(sid == NS - 1)
    def _():
        pltpu.sync_copy(acc_sh.at[pl.ds(sid * RPS, RPS_LAST)],
                        out_hbm.at[cid, pl.ds(sid * RPS, RPS_LAST)])


@functools.partial(
    pl.kernel,
    out_type=jax.ShapeDtypeStruct((NC, N, 128), jnp.float32),
    mesh=_MESH,
    scratch_types=[
        [pltpu.VMEM((C2,), jnp.int32)] * RS,
        [pltpu.VMEM((C2,), jnp.int32)] * RS,
        [pltpu.VMEM((C2, 128), jnp.float32)] * RS,
        pltpu.VMEM_SHARED((N_ACC, 128), jnp.float32),
        [pltpu.SemaphoreType.DMA] * RS,
        [pltpu.SemaphoreType.DMA] * RS,
        [pltpu.SemaphoreType.DMA] * RS,
    ],
)
def _scatter_kernel(y_hbm, src_hbm, dst_hbm, zeros_hbm, out_hbm,
                    srcv, dstv, rows, acc_sh, isem, gsem, ssem):
    cid = lax.axis_index("c")
    sid = lax.axis_index("s")
    wid = sid * NC + cid
    ebase = wid * EWP

    def idx_async(c, b):
        base = ebase + c * C2
        pltpu.async_copy(src_hbm.at[pl.ds(base, C2)], srcv[b], isem[b])
        pltpu.async_copy(dst_hbm.at[pl.ds(base, C2)], dstv[b], isem[b])

    def idx_wait(c, b):
        base = ebase + c * C2
        pltpu.make_async_copy(src_hbm.at[pl.ds(base, C2)], srcv[b], isem[b]).wait()
        pltpu.make_async_copy(dst_hbm.at[pl.ds(base, C2)], dstv[b], isem[b]).wait()

    def gather(b):
        pltpu.async_copy(y_hbm.at[srcv[b]], rows[b], gsem[b])

    def gather_wait(b):
        pltpu.make_async_copy(y_hbm.at[srcv[b]], rows[b], gsem[b]).wait()

    def scatter(b):
        pltpu.async_copy(rows[b], acc_sh.at[dstv[b]], ssem[b], add=True)

    def scatter_wait(b):
        pltpu.make_async_copy(rows[b], acc_sh.at[dstv[b]], ssem[b]).wait()

    _init_acc(sid, zeros_hbm, acc_sh)
    for b in range(RS):
        idx_async(b, b)
    for b in range(RS):
        idx_wait(b, b)
        gather(b)
    plsc.subcore_barrier()

    def body(g, _):
        cn = RS * g
        for b in range(RS):
            gather_wait(b)
            scatter(b)
        for b in range(RS):
            scatter_wait(b)
            idx_async(cn + b, b)
        for b in range(RS):
            idx_wait(cn + b, b)
            gather(b)
        return 0

    lax.fori_loop(1, NGRP2, body, 0)
    for b in range(RS):
        gather_wait(b)
        scatter(b)
    for b in range(RS):
        scatter_wait(b)
    plsc.subcore_barrier()
    _copy_out(cid, sid, acc_sh, out_hbm)


@functools.partial(
    pl.kernel,
    out_type=jax.ShapeDtypeStruct((NC, N, 128), jnp.float32),
    mesh=_MESH,
    scratch_types=[
        [pltpu.VMEM((C,), jnp.int32)] * 4,
        pltpu.VMEM((C, 128), jnp.float32),
        pltpu.VMEM_SHARED((N_ACC, 128), jnp.float32),
        [pltpu.SemaphoreType.DMA] * 4,
        [pltpu.SemaphoreType.DMA] * 4,
    ],
)
def _deg_kernel(dst_hbm, ones_hbm, zeros_hbm, out_hbm,
                dstv, ones_v, acc_sh, isem, ssem):
    cid = lax.axis_index("c")
    sid = lax.axis_index("s")
    wid = sid * NC + cid
    ebase = wid * EWP

    def idx_async(c, b):
        pltpu.async_copy(dst_hbm.at[pl.ds(ebase + c * C, C)], dstv[b], isem[b])

    def idx_wait(c, b):
        pltpu.make_async_copy(dst_hbm.at[pl.ds(ebase + c * C, C)],
                              dstv[b], isem[b]).wait()

    def scatter(b):
        pltpu.async_copy(ones_v, acc_sh.at[dstv[b]], ssem[b], add=True)

    def scatter_wait(b):
        pltpu.make_async_copy(ones_v, acc_sh.at[dstv[b]], ssem[b]).wait()

    _init_acc(sid, zeros_hbm, acc_sh)
    pltpu.sync_copy(ones_hbm, ones_v)
    for b in range(4):
        idx_async(b, b)
    plsc.subcore_barrier()
    for b in range(4):
        idx_wait(b, b)
        scatter(b)

    def body(g, _):
        cn = 4 * g
        for b in range(4):
            scatter_wait(b)
            idx_async(cn + b, b)
        for b in range(4):
            idx_wait(cn + b, b)
            scatter(b)
        return 0

    lax.fori_loop(1, CHUNKS // 4, body, 0)
    for b in range(4):
        scatter_wait(b)
    plsc.subcore_barrier()
    _copy_out(cid, sid, acc_sh, out_hbm)


_ROWS = 1000
_GRID = N // _ROWS


def _elu(v):
    return jnp.where(v > 0, v, jnp.exp(jnp.minimum(v, 0.0)) - 1.0)


def _row_gate():
    base = pl.program_id(0) * _ROWS
    rows = base + lax.broadcasted_iota(jnp.int32, (_ROWS, 1), 0)
    return (rows >= NW * PAD).astype(jnp.float32)


def _t1_body(x_ref, w_ref, degp_ref, y_ref, dinv_ref):
    deg = degp_ref[0, :, 0:1] + degp_ref[1, :, 0:1] + _row_gate()
    dinv = lax.rsqrt(deg)
    dinv_ref[...] = dinv
    y_ref[...] = jnp.dot(x_ref[...], w_ref[...],
                         preferred_element_type=jnp.float32) * dinv


def _t1(x, W1, degp):
    return pl.pallas_call(
        _t1_body,
        grid=(_GRID,),
        in_specs=[
            pl.BlockSpec((_ROWS, 128), lambda i: (i, 0)),
            pl.BlockSpec((128, 128), lambda i: (0, 0)),
            pl.BlockSpec((2, _ROWS, 128), lambda i: (0, i, 0)),
        ],
        out_specs=[
            pl.BlockSpec((_ROWS, 128), lambda i: (i, 0)),
            pl.BlockSpec((_ROWS, 1), lambda i: (i, 0)),
        ],
        out_shape=[
            jax.ShapeDtypeStruct((N, 128), jnp.float32),
            jax.ShapeDtypeStruct((N, 1), jnp.float32),
        ],
    )(x, W1, degp)


def _t2_body(p0_ref, p1_ref, y_ref, dinv_ref, b_ref, w_ref, o_ref):
    agg = p0_ref[...] + p1_ref[...] + y_ref[...] * _row_gate()
    h = _elu(agg * dinv_ref[...] + b_ref[...])
    o_ref[...] = jnp.dot(h, w_ref[...],
                         preferred_element_type=jnp.float32) * dinv_ref[...]


def _t2(p0, p1, y1, dinv, b1, W2):
    return pl.pallas_call(
        _t2_body,
        grid=(_GRID,),
        in_specs=[
            pl.BlockSpec((_ROWS, 128), lambda i: (i, 0)),
            pl.BlockSpec((_ROWS, 128), lambda i: (i, 0)),
            pl.BlockSpec((_ROWS, 128), lambda i: (i, 0)),
            pl.BlockSpec((_ROWS, 1), lambda i: (i, 0)),
            pl.BlockSpec((1, 128), lambda i: (0, 0)),
            pl.BlockSpec((128, 128), lambda i: (0, 0)),
        ],
        out_specs=pl.BlockSpec((_ROWS, 128), lambda i: (i, 0)),
        out_shape=jax.ShapeDtypeStruct((N, 128), jnp.float32),
    )(p0, p1, y1, dinv, b1, W2)


def _t3_body(p0_ref, p1_ref, y_ref, dinv_ref, b_ref, w_ref, bl_ref, o_ref):
    agg = p0_ref[...] + p1_ref[...] + y_ref[...] * _row_gate()
    h = _elu(agg * dinv_ref[...] + b_ref[...])
    o_ref[...] = jnp.dot(h, w_ref[...],
                         preferred_element_type=jnp.float32) + bl_ref[...]


def _t3(p0, p1, y2, dinv, b2, Wl, bl):
    return pl.pallas_call(
        _t3_body,
        grid=(_GRID,),
        in_specs=[
            pl.BlockSpec((_ROWS, 128), lambda i: (i, 0)),
            pl.BlockSpec((_ROWS, 128), lambda i: (i, 0)),
            pl.BlockSpec((_ROWS, 128), lambda i: (i, 0)),
            pl.BlockSpec((_ROWS, 1), lambda i: (i, 0)),
            pl.BlockSpec((1, 128), lambda i: (0, 0)),
            pl.BlockSpec((128, 64), lambda i: (0, 0)),
            pl.BlockSpec((1, 64), lambda i: (0, 0)),
        ],
        out_specs=pl.BlockSpec((_ROWS, 64), lambda i: (i, 0)),
        out_shape=jax.ShapeDtypeStruct((N, 64), jnp.float32),
    )(p0, p1, y2, dinv, b2, Wl, bl)


def kernel(x, edge_index, W1, b1, W2, b2, Wl, bl):
    src = edge_index[0].astype(jnp.int32)
    dst = edge_index[1].astype(jnp.int32)
    pad_dst = jnp.arange(NW * PAD, dtype=jnp.int32).reshape(NW, PAD)
    srcp = jnp.concatenate([src.reshape(NW, EW), pad_dst], axis=1).reshape(NW * EWP)
    dstp = jnp.concatenate([dst.reshape(NW, EW), pad_dst], axis=1).reshape(NW * EWP)
    ones = jnp.ones((C, 128), jnp.float32)
    zeros = jnp.zeros((RPS, 128), jnp.float32)

    degp = _deg_kernel(dstp, ones, zeros)

    y1, dinv = _t1(x, W1, degp)
    p1 = _scatter_kernel(y1, srcp, dstp, zeros)
    y2 = _t2(p1[0], p1[1], y1, dinv, b1.reshape(1, 128), W2)
    p2 = _scatter_kernel(y2, srcp, dstp, zeros)
    out = _t3(p2[0], p2[1], y2, dinv, b2.reshape(1, 128), Wl,
              bl.reshape(1, 64))
    return out


# speedup vs baseline: 1.0853x; 1.0320x over previous
import functools

import jax
import jax.numpy as jnp
from jax import lax
from jax.experimental import pallas as pl
from jax.experimental.pallas import tpu as pltpu
from jax.experimental.pallas import tpu_sc as plsc

N = 10000
E = 320000
NC = 2
NS = 16
NW = NC * NS
EW = E // NW
C = 128
CHUNKS = 80
EWP = C * CHUNKS
C2 = 32
RS = 10
CHUNKS2 = EWP // C2
NGRP2 = CHUNKS2 // RS
PAD = EWP - EW
NBUF = 2
NGRP = CHUNKS // NBUF
N_ACC = N
RPS = 632
RPS_LAST = N - 15 * RPS

_MESH = plsc.VectorSubcoreMesh(core_axis_name="c", subcore_axis_name="s")


def _init_acc(sid, zeros_hbm, acc_sh):
    @pl.when(sid < NS - 1)
    def _():
        pltpu.sync_copy(zeros_hbm, acc_sh.at[pl.ds(sid * RPS, RPS)])

    @pl.when(sid == NS - 1)
    def _():
        pltpu.sync_copy(zeros_hbm.at[pl.ds(0, RPS_LAST)],
                        acc_sh.at[pl.ds(sid * RPS, RPS_LAST)])


def _copy_out(cid, sid, acc_sh, out_hbm):
    @pl.when(sid < NS - 1)
    def _():
        pltpu.sync_copy(acc_sh.at[pl.ds(sid * RPS, RPS)],
                        out_hbm.at[cid, pl.ds(sid * RPS, RPS)])

    @pl.when(sid == NS - 1)
    def _():
        pltpu.sync_copy(acc_sh.at[pl.ds(sid * RPS, RPS_LAST)],
                        out_hbm.at[cid, pl.ds(sid * RPS, RPS_LAST)])


@functools.partial(
    pl.kernel,
    out_type=jax.ShapeDtypeStruct((NC, N, 128), jnp.float32),
    mesh=_MESH,
    scratch_types=[
        [pltpu.VMEM((C2,), jnp.int32)] * RS,
        [pltpu.VMEM((C2,), jnp.int32)] * RS,
        [pltpu.VMEM((C2, 128), jnp.float32)] * RS,
        pltpu.VMEM_SHARED((N_ACC, 128), jnp.float32),
        [pltpu.SemaphoreType.DMA] * RS,
        [pltpu.SemaphoreType.DMA] * RS,
        [pltpu.SemaphoreType.DMA] * RS,
    ],
)
def _scatter_kernel(y_hbm, src_hbm, dst_hbm, zeros_hbm, out_hbm,
                    srcv, dstv, rows, acc_sh, isem, gsem, ssem):
    cid = lax.axis_index("c")
    sid = lax.axis_index("s")
    wid = sid * NC + cid
    ebase = wid * EWP

    def idx_async(c, b):
        base = ebase + c * C2
        pltpu.async_copy(src_hbm.at[pl.ds(base, C2)], srcv[b], isem[b])
        pltpu.async_copy(dst_hbm.at[pl.ds(base, C2)], dstv[b], isem[b])

    def idx_wait(c, b):
        base = ebase + c * C2
        pltpu.make_async_copy(src_hbm.at[pl.ds(base, C2)], srcv[b], isem[b]).wait()
        pltpu.make_async_copy(dst_hbm.at[pl.ds(base, C2)], dstv[b], isem[b]).wait()

    def gather(b):
        pltpu.async_copy(y_hbm.at[srcv[b]], rows[b], gsem[b])

    def gather_wait(b):
        pltpu.make_async_copy(y_hbm.at[srcv[b]], rows[b], gsem[b]).wait()

    def scatter(b):
        pltpu.async_copy(rows[b], acc_sh.at[dstv[b]], ssem[b], add=True)

    def scatter_wait(b):
        pltpu.make_async_copy(rows[b], acc_sh.at[dstv[b]], ssem[b]).wait()

    _init_acc(sid, zeros_hbm, acc_sh)
    for b in range(RS):
        idx_async(b, b)
    for b in range(RS):
        idx_wait(b, b)
        gather(b)
    plsc.subcore_barrier()

    def body(g, _):
        cn = RS * g
        for b in range(RS):
            gather_wait(b)
            scatter(b)
        for b in range(RS):
            scatter_wait(b)
            idx_async(cn + b, b)
        for b in range(RS):
            idx_wait(cn + b, b)
            gather(b)
        return 0

    lax.fori_loop(1, NGRP2, body, 0)
    for b in range(RS):
        gather_wait(b)
        scatter(b)
    for b in range(RS):
        scatter_wait(b)
    plsc.subcore_barrier()
    _copy_out(cid, sid, acc_sh, out_hbm)


@functools.partial(
    pl.kernel,
    out_type=jax.ShapeDtypeStruct((NC, N, 128), jnp.float32),
    mesh=_MESH,
    scratch_types=[
        [pltpu.VMEM((C,), jnp.int32)] * 4,
        pltpu.VMEM((C, 128), jnp.float32),
        pltpu.VMEM_SHARED((N_ACC, 128), jnp.float32),
        [pltpu.SemaphoreType.DMA] * 4,
        [pltpu.SemaphoreType.DMA] * 4,
    ],
)
def _deg_kernel(dst_hbm, ones_hbm, zeros_hbm, out_hbm,
                dstv, ones_v, acc_sh, isem, ssem):
    cid = lax.axis_index("c")
    sid = lax.axis_index("s")
    wid = sid * NC + cid
    ebase = wid * EWP

    def idx_async(c, b):
        pltpu.async_copy(dst_hbm.at[pl.ds(ebase + c * C, C)], dstv[b], isem[b])

    def idx_wait(c, b):
        pltpu.make_async_copy(dst_hbm.at[pl.ds(ebase + c * C, C)],
                              dstv[b], isem[b]).wait()

    def scatter(b):
        pltpu.async_copy(ones_v, acc_sh.at[dstv[b]], ssem[b], add=True)

    def scatter_wait(b):
        pltpu.make_async_copy(ones_v, acc_sh.at[dstv[b]], ssem[b]).wait()

    _init_acc(sid, zeros_hbm, acc_sh)
    pltpu.sync_copy(ones_hbm, ones_v)
    for b in range(4):
        idx_async(b, b)
    plsc.subcore_barrier()
    for b in range(4):
        idx_wait(b, b)
        scatter(b)

    def body(g, _):
        cn = 4 * g
        for b in range(4):
            scatter_wait(b)
            idx_async(cn + b, b)
        for b in range(4):
            idx_wait(cn + b, b)
            scatter(b)
        return 0

    lax.fori_loop(1, CHUNKS // 4, body, 0)
    for b in range(4):
        scatter_wait(b)
    plsc.subcore_barrier()
    _copy_out(cid, sid, acc_sh, out_hbm)


_ROWS = 1000
_GRID = N // _ROWS


def _elu(v):
    return jnp.where(v > 0, v, jnp.exp(jnp.minimum(v, 0.0)) - 1.0)


def _row_gate():
    base = pl.program_id(0) * _ROWS
    rows = base + lax.broadcasted_iota(jnp.int32, (_ROWS, 1), 0)
    return (rows >= NW * PAD).astype(jnp.float32)


def _t1_body(x_ref, w_ref, degp_ref, y_ref, dinv_ref):
    deg = degp_ref[0, :, 0:1] + degp_ref[1, :, 0:1] + _row_gate()
    dinv = lax.rsqrt(deg)
    dinv_ref[...] = dinv
    y_ref[...] = jnp.dot(x_ref[...], w_ref[...],
                         preferred_element_type=jnp.float32) * dinv


def _t1(x, W1, degp):
    return pl.pallas_call(
        _t1_body,
        grid=(_GRID,),
        in_specs=[
            pl.BlockSpec((_ROWS, 128), lambda i: (i, 0)),
            pl.BlockSpec((128, 128), lambda i: (0, 0)),
            pl.BlockSpec((2, _ROWS, 128), lambda i: (0, i, 0)),
        ],
        out_specs=[
            pl.BlockSpec((_ROWS, 128), lambda i: (i, 0)),
            pl.BlockSpec((_ROWS, 1), lambda i: (i, 0)),
        ],
        out_shape=[
            jax.ShapeDtypeStruct((N, 128), jnp.float32),
            jax.ShapeDtypeStruct((N, 1), jnp.float32),
        ],
    )(x, W1, degp)


def _t2_body(p0_ref, p1_ref, y_ref, dinv_ref, b_ref, w_ref, o_ref):
    agg = p0_ref[...] + p1_ref[...] + y_ref[...] * _row_gate()
    h = _elu(agg * dinv_ref[...] + b_ref[...])
    o_ref[...] = jnp.dot(h, w_ref[...],
                         preferred_element_type=jnp.float32) * dinv_ref[...]


def _t2(p0, p1, y1, dinv, b1, W2):
    return pl.pallas_call(
        _t2_body,
        grid=(_GRID,),
        in_specs=[
            pl.BlockSpec((_ROWS, 128), lambda i: (i, 0)),
            pl.BlockSpec((_ROWS, 128), lambda i: (i, 0)),
            pl.BlockSpec((_ROWS, 128), lambda i: (i, 0)),
            pl.BlockSpec((_ROWS, 1), lambda i: (i, 0)),
            pl.BlockSpec((1, 128), lambda i: (0, 0)),
            pl.BlockSpec((128, 128), lambda i: (0, 0)),
        ],
        out_specs=pl.BlockSpec((_ROWS, 128), lambda i: (i, 0)),
        out_shape=jax.ShapeDtypeStruct((N, 128), jnp.float32),
    )(p0, p1, y1, dinv, b1, W2)


def _t3_body(p0_ref, p1_ref, y_ref, dinv_ref, b_ref, w_ref, bl_ref, o_ref):
    agg = p0_ref[...] + p1_ref[...] + y_ref[...] * _row_gate()
    h = _elu(agg * dinv_ref[...] + b_ref[...])
    o_ref[...] = jnp.dot(h, w_ref[...],
                         preferred_element_type=jnp.float32) + bl_ref[...]


def _t3(p0, p1, y2, dinv, b2, Wl, bl):
    return pl.pallas_call(
        _t3_body,
        grid=(_GRID,),
        in_specs=[
            pl.BlockSpec((_ROWS, 128), lambda i: (i, 0)),
            pl.BlockSpec((_ROWS, 128), lambda i: (i, 0)),
            pl.BlockSpec((_ROWS, 128), lambda i: (i, 0)),
            pl.BlockSpec((_ROWS, 1), lambda i: (i, 0)),
            pl.BlockSpec((1, 128), lambda i: (0, 0)),
            pl.BlockSpec((128, 64), lambda i: (0, 0)),
            pl.BlockSpec((1, 64), lambda i: (0, 0)),
        ],
        out_specs=pl.BlockSpec((_ROWS, 64), lambda i: (i, 0)),
        out_shape=jax.ShapeDtypeStruct((N, 64), jnp.float32),
    )(p0, p1, y2, dinv, b2, Wl, bl)


def kernel(x, edge_index, W1, b1, W2, b2, Wl, bl):
    src = edge_index[0].astype(jnp.int32)
    dst = edge_index[1].astype(jnp.int32)
    pad_dst = jnp.arange(NW * PAD, dtype=jnp.int32).reshape(NW, PAD)
    srcp = jnp.concatenate([src.reshape(NW, EW), pad_dst], axis=1).reshape(NW * EWP)
    dstp = jnp.concatenate([dst.reshape(NW, EW), pad_dst], axis=1).reshape(NW * EWP)
    ones = jnp.ones((C, 128), jnp.float32)
    zeros = jnp.zeros((RPS, 128), jnp.float32)

    degp = _deg_kernel(dstp, ones, zeros)

    y1, dinv = _t1(x, W1, degp)
    p1 = _scatter_kernel(y1, srcp, dstp, zeros)
    y2 = _t2(p1[0], p1[1], y1, dinv, b1.reshape(1, 128), W2)
    p2 = _scatter_kernel(y2, srcp, dstp, zeros)
    out = _t3(p2[0], p2[1], y2, dinv, b2.reshape(1, 128), Wl,
              bl.reshape(1, 64))
    return out


# speedup vs baseline: 1.0861x; 1.0007x over previous
import functools

import jax
import jax.numpy as jnp
from jax import lax
from jax.experimental import pallas as pl
from jax.experimental.pallas import tpu as pltpu
from jax.experimental.pallas import tpu_sc as plsc

N = 10000
E = 320000
NC = 2
NS = 16
NW = NC * NS
EW = E // NW
C = 128
CHUNKS = 80
EWP = C * CHUNKS
C2 = 32
RS = 10
CHUNKS2 = EWP // C2
NGRP2 = CHUNKS2 // RS
PAD = EWP - EW
NBUF = 2
NGRP = CHUNKS // NBUF
N_ACC = N
RPS = 632
RPS_LAST = N - 15 * RPS

_MESH = plsc.VectorSubcoreMesh(core_axis_name="c", subcore_axis_name="s")


def _init_acc(sid, zeros_hbm, acc_sh):
    @pl.when(sid < NS - 1)
    def _():
        pltpu.sync_copy(zeros_hbm, acc_sh.at[pl.ds(sid * RPS, RPS)])

    @pl.when(sid == NS - 1)
    def _():
        pltpu.sync_copy(zeros_hbm.at[pl.ds(0, RPS_LAST)],
                        acc_sh.at[pl.ds(sid * RPS, RPS_LAST)])


def _copy_out(cid, sid, acc_sh, out_hbm):
    @pl.when(sid < NS - 1)
    def _():
        pltpu.sync_copy(acc_sh.at[pl.ds(sid * RPS, RPS)],
                        out_hbm.at[cid, pl.ds(sid * RPS, RPS)])

    @pl.when(sid == NS - 1)
    def _():
        pltpu.sync_copy(acc_sh.at[pl.ds(sid * RPS, RPS_LAST)],
                        out_hbm.at[cid, pl.ds(sid * RPS, RPS_LAST)])


@functools.partial(
    pl.kernel,
    out_type=jax.ShapeDtypeStruct((NC, N, 128), jnp.float32),
    mesh=_MESH,
    scratch_types=[
        [pltpu.VMEM((C2,), jnp.int32)] * RS,
        [pltpu.VMEM((C2,), jnp.int32)] * RS,
        [pltpu.VMEM((C2, 128), jnp.float32)] * RS,
        pltpu.VMEM_SHARED((N_ACC, 128), jnp.float32),
        [pltpu.SemaphoreType.DMA] * RS,
        [pltpu.SemaphoreType.DMA] * RS,
        [pltpu.SemaphoreType.DMA] * RS,
    ],
)
def _scatter_kernel(y_hbm, src_hbm, dst_hbm, zeros_hbm, out_hbm,
                    srcv, dstv, rows, acc_sh, isem, gsem, ssem):
    cid = lax.axis_index("c")
    sid = lax.axis_index("s")
    wid = sid * NC + cid
    ebase = wid * EWP

    def idx_async(c, b):
        base = ebase + c * C2
        pltpu.async_copy(src_hbm.at[pl.ds(base, C2)], srcv[b], isem[b])
        pltpu.async_copy(dst_hbm.at[pl.ds(base, C2)], dstv[b], isem[b])

    def idx_wait(c, b):
        base = ebase + c * C2
        pltpu.make_async_copy(src_hbm.at[pl.ds(base, C2)], srcv[b], isem[b]).wait()
        pltpu.make_async_copy(dst_hbm.at[pl.ds(base, C2)], dstv[b], isem[b]).wait()

    def gather(b):
        pltpu.async_copy(y_hbm.at[srcv[b]], rows[b], gsem[b])

    def gather_wait(b):
        pltpu.make_async_copy(y_hbm.at[srcv[b]], rows[b], gsem[b]).wait()

    def scatter(b):
        pltpu.async_copy(rows[b], acc_sh.at[dstv[b]], ssem[b], add=True)

    def scatter_wait(b):
        pltpu.make_async_copy(rows[b], acc_sh.at[dstv[b]], ssem[b]).wait()

    _init_acc(sid, zeros_hbm, acc_sh)
    for b in range(RS):
        idx_async(b, b)
    for b in range(RS):
        idx_wait(b, b)
        gather(b)
    plsc.subcore_barrier()

    def body(g, _):
        cn = RS * g
        for b in range(RS):
            gather_wait(b)
            scatter(b)
        for b in range(RS):
            scatter_wait(b)
            idx_async(cn + b, b)
        for b in range(RS):
            idx_wait(cn + b, b)
            gather(b)
        return 0

    lax.fori_loop(1, NGRP2, body, 0)
    for b in range(RS):
        gather_wait(b)
        scatter(b)
    for b in range(RS):
        scatter_wait(b)
    plsc.subcore_barrier()
    _copy_out(cid, sid, acc_sh, out_hbm)


@functools.partial(
    pl.kernel,
    out_type=jax.ShapeDtypeStruct((NC, N, 128), jnp.float32),
    mesh=_MESH,
    scratch_types=[
        [pltpu.VMEM((C,), jnp.int32)] * 8,
        pltpu.VMEM((C, 128), jnp.float32),
        pltpu.VMEM_SHARED((N_ACC, 128), jnp.float32),
        [pltpu.SemaphoreType.DMA] * 8,
        [pltpu.SemaphoreType.DMA] * 8,
    ],
)
def _deg_kernel(dst_hbm, ones_hbm, zeros_hbm, out_hbm,
                dstv, ones_v, acc_sh, isem, ssem):
    cid = lax.axis_index("c")
    sid = lax.axis_index("s")
    wid = sid * NC + cid
    ebase = wid * EWP

    def idx_async(c, b):
        pltpu.async_copy(dst_hbm.at[pl.ds(ebase + c * C, C)], dstv[b], isem[b])

    def idx_wait(c, b):
        pltpu.make_async_copy(dst_hbm.at[pl.ds(ebase + c * C, C)],
                              dstv[b], isem[b]).wait()

    def scatter(b):
        pltpu.async_copy(ones_v, acc_sh.at[dstv[b]], ssem[b], add=True)

    def scatter_wait(b):
        pltpu.make_async_copy(ones_v, acc_sh.at[dstv[b]], ssem[b]).wait()

    _init_acc(sid, zeros_hbm, acc_sh)
    pltpu.sync_copy(ones_hbm, ones_v)
    for b in range(8):
        idx_async(b, b)
    plsc.subcore_barrier()
    for b in range(8):
        idx_wait(b, b)
        scatter(b)

    def body(g, _):
        cn = 8 * g
        for b in range(8):
            scatter_wait(b)
            idx_async(cn + b, b)
        for b in range(8):
            idx_wait(cn + b, b)
            scatter(b)
        return 0

    lax.fori_loop(1, CHUNKS // 8, body, 0)
    for b in range(8):
        scatter_wait(b)
    plsc.subcore_barrier()
    _copy_out(cid, sid, acc_sh, out_hbm)


_ROWS = 1000
_GRID = N // _ROWS


def _elu(v):
    return jnp.where(v > 0, v, jnp.exp(jnp.minimum(v, 0.0)) - 1.0)


def _row_gate():
    base = pl.program_id(0) * _ROWS
    rows = base + lax.broadcasted_iota(jnp.int32, (_ROWS, 1), 0)
    return (rows >= NW * PAD).astype(jnp.float32)


def _t1_body(x_ref, w_ref, degp_ref, y_ref, dinv_ref):
    deg = degp_ref[0, :, 0:1] + degp_ref[1, :, 0:1] + _row_gate()
    dinv = lax.rsqrt(deg)
    dinv_ref[...] = dinv
    y_ref[...] = jnp.dot(x_ref[...], w_ref[...],
                         preferred_element_type=jnp.float32) * dinv


def _t1(x, W1, degp):
    return pl.pallas_call(
        _t1_body,
        grid=(_GRID,),
        in_specs=[
            pl.BlockSpec((_ROWS, 128), lambda i: (i, 0)),
            pl.BlockSpec((128, 128), lambda i: (0, 0)),
            pl.BlockSpec((2, _ROWS, 128), lambda i: (0, i, 0)),
        ],
        out_specs=[
            pl.BlockSpec((_ROWS, 128), lambda i: (i, 0)),
            pl.BlockSpec((_ROWS, 1), lambda i: (i, 0)),
        ],
        out_shape=[
            jax.ShapeDtypeStruct((N, 128), jnp.float32),
            jax.ShapeDtypeStruct((N, 1), jnp.float32),
        ],
    )(x, W1, degp)


def _t2_body(p0_ref, p1_ref, y_ref, dinv_ref, b_ref, w_ref, o_ref):
    agg = p0_ref[...] + p1_ref[...] + y_ref[...] * _row_gate()
    h = _elu(agg * dinv_ref[...] + b_ref[...])
    o_ref[...] = jnp.dot(h, w_ref[...],
                         preferred_element_type=jnp.float32) * dinv_ref[...]


def _t2(p0, p1, y1, dinv, b1, W2):
    return pl.pallas_call(
        _t2_body,
        grid=(_GRID,),
        in_specs=[
            pl.BlockSpec((_ROWS, 128), lambda i: (i, 0)),
            pl.BlockSpec((_ROWS, 128), lambda i: (i, 0)),
            pl.BlockSpec((_ROWS, 128), lambda i: (i, 0)),
            pl.BlockSpec((_ROWS, 1), lambda i: (i, 0)),
            pl.BlockSpec((1, 128), lambda i: (0, 0)),
            pl.BlockSpec((128, 128), lambda i: (0, 0)),
        ],
        out_specs=pl.BlockSpec((_ROWS, 128), lambda i: (i, 0)),
        out_shape=jax.ShapeDtypeStruct((N, 128), jnp.float32),
    )(p0, p1, y1, dinv, b1, W2)


def _t3_body(p0_ref, p1_ref, y_ref, dinv_ref, b_ref, w_ref, bl_ref, o_ref):
    agg = p0_ref[...] + p1_ref[...] + y_ref[...] * _row_gate()
    h = _elu(agg * dinv_ref[...] + b_ref[...])
    o_ref[...] = jnp.dot(h, w_ref[...],
                         preferred_element_type=jnp.float32) + bl_ref[...]


def _t3(p0, p1, y2, dinv, b2, Wl, bl):
    return pl.pallas_call(
        _t3_body,
        grid=(_GRID,),
        in_specs=[
            pl.BlockSpec((_ROWS, 128), lambda i: (i, 0)),
            pl.BlockSpec((_ROWS, 128), lambda i: (i, 0)),
            pl.BlockSpec((_ROWS, 128), lambda i: (i, 0)),
            pl.BlockSpec((_ROWS, 1), lambda i: (i, 0)),
            pl.BlockSpec((1, 128), lambda i: (0, 0)),
            pl.BlockSpec((128, 64), lambda i: (0, 0)),
            pl.BlockSpec((1, 64), lambda i: (0, 0)),
        ],
        out_specs=pl.BlockSpec((_ROWS, 64), lambda i: (i, 0)),
        out_shape=jax.ShapeDtypeStruct((N, 64), jnp.float32),
    )(p0, p1, y2, dinv, b2, Wl, bl)


def kernel(x, edge_index, W1, b1, W2, b2, Wl, bl):
    src = edge_index[0].astype(jnp.int32)
    dst = edge_index[1].astype(jnp.int32)
    pad_dst = jnp.arange(NW * PAD, dtype=jnp.int32).reshape(NW, PAD)
    srcp = jnp.concatenate([src.reshape(NW, EW), pad_dst], axis=1).reshape(NW * EWP)
    dstp = jnp.concatenate([dst.reshape(NW, EW), pad_dst], axis=1).reshape(NW * EWP)
    ones = jnp.ones((C, 128), jnp.float32)
    zeros = jnp.zeros((RPS, 128), jnp.float32)

    degp = _deg_kernel(dstp, ones, zeros)

    y1, dinv = _t1(x, W1, degp)
    p1 = _scatter_kernel(y1, srcp, dstp, zeros)
    y2 = _t2(p1[0], p1[1], y1, dinv, b1.reshape(1, 128), W2)
    p2 = _scatter_kernel(y2, srcp, dstp, zeros)
    out = _t3(p2[0], p2[1], y2, dinv, b2.reshape(1, 128), Wl,
              bl.reshape(1, 64))
    return out
